# Initial kernel scaffold; baseline (speedup 1.0000x reference)
import jax
import jax.numpy as jnp
from jax.experimental import pallas as pl


def kernel(input, edge, edge_embed, a, a_2):
    raise NotImplementedError("write your pallas kernel here")


# speedup vs baseline: 2.4181x; 2.4181x over previous
import functools

import jax
import jax.numpy as jnp
from jax import lax
from jax.experimental import pallas as pl
from jax.experimental.pallas import tpu as pltpu
from jax.experimental.pallas import tpu_sc as plsc

N = 10000
E = 320000
F = 128
NR = 16
ALPHA = 0.2

NC = 2
NS = 16
NW = NC * NS
EPW = E // NW
B = 80
NB = EPW // B
WEE = 32
RPT = 624
NTAIL = N - RPT * NS


def _sc_body(e0_hbm, e1_hbm, ee_hbm, se_hbm, v_hbm, su_hbm, sv_hbm,
             outV_hbm, outE_hbm,
             e0_b, e1_b, ee_b, se_b, sug_b, svg_b, w_b, rows_b, wee_b,
             accV, accEW, sem, sem2, sem3):
    c = lax.axis_index("c")
    s = lax.axis_index("s")
    wid = c * NS + s

    zv = jnp.zeros((16,), jnp.float32)

    def _zrow(e, _):
        for j in range(F // 16):
            rows_b[e, pl.ds(j * 16, 16)] = zv
        for j in range(WEE // 16):
            wee_b[e, pl.ds(j * 16, 16)] = zv
        return 0
    lax.fori_loop(0, B, _zrow, 0)

    for k in range(RPT // B):
        r0 = s * RPT + k * B
        pltpu.sync_copy(rows_b, accV.at[pl.ds(r0, B), :])
        pltpu.sync_copy(wee_b, accEW.at[pl.ds(r0, B), :])
    r0 = s * RPT + (RPT // B) * B
    rem = RPT - (RPT // B) * B
    if rem:
        pltpu.sync_copy(rows_b.at[pl.ds(0, rem), :], accV.at[pl.ds(r0, rem), :])
        pltpu.sync_copy(wee_b.at[pl.ds(0, rem), :], accEW.at[pl.ds(r0, rem), :])

    @pl.when(s == NS - 1)
    def _ztail():
        pltpu.sync_copy(rows_b.at[pl.ds(0, NTAIL), :],
                        accV.at[pl.ds(RPT * NS, NTAIL), :])
        pltpu.sync_copy(wee_b.at[pl.ds(0, NTAIL), :],
                        accEW.at[pl.ds(RPT * NS, NTAIL), :])

    plsc.subcore_barrier()

    lanes = lax.iota(jnp.int32, 16)
    wcol = jnp.full((16,), NR, jnp.int32)

    def _batch(i, _):
        off = pl.multiple_of(wid * EPW + i * B, B)
        pltpu.sync_copy(e0_hbm.at[pl.ds(off, B)], e0_b)
        pltpu.sync_copy(e1_hbm.at[pl.ds(off, B)], e1_b)
        pltpu.sync_copy(ee_hbm.at[pl.ds(off, B), :], ee_b)
        pltpu.sync_copy(se_hbm.at[pl.ds(off, B)], se_b)
        gat = pltpu.async_copy(v_hbm.at[e1_b], rows_b, sem)
        gsu = pltpu.async_copy(su_hbm.at[e0_b], sug_b, sem2)
        gsv = pltpu.async_copy(sv_hbm.at[e1_b], svg_b, sem3)
        gsu.wait()
        gsv.wait()

        for j in range(B // 16):
            sl = pl.ds(j * 16, 16)
            z = sug_b[sl] + svg_b[sl] + se_b[sl]
            w = jnp.exp(jnp.minimum(-z, (-ALPHA) * z))
            w_b[sl] = w
            plsc.store_scatter(wee_b, [lanes + j * 16, wcol], w)

        gat.wait()

        def _edge(e, _):
            wv = plsc.load_gather(w_b, [jnp.full((16,), e, jnp.int32)])
            for j in range(F // 16):
                sl = pl.ds(j * 16, 16)
                rows_b[e, sl] = wv * rows_b[e, sl]
            wee_b[e, pl.ds(0, 16)] = wv * ee_b[e, :]
            return 0
        lax.fori_loop(0, B, _edge, 0)

        pltpu.sync_copy(rows_b, accV.at[e0_b], add=True)
        pltpu.sync_copy(wee_b, accEW.at[e0_b], add=True)
        return 0

    lax.fori_loop(0, NB, _batch, 0)

    plsc.subcore_barrier()

    for k in range(RPT // B):
        r0 = s * RPT + k * B
        pltpu.sync_copy(accV.at[pl.ds(r0, B), :], outV_hbm.at[c, pl.ds(r0, B), :])
        pltpu.sync_copy(accEW.at[pl.ds(r0, B), :], outE_hbm.at[c, pl.ds(r0, B), :])
    r0 = s * RPT + (RPT // B) * B
    if rem:
        pltpu.sync_copy(accV.at[pl.ds(r0, rem), :],
                        outV_hbm.at[c, pl.ds(r0, rem), :])
        pltpu.sync_copy(accEW.at[pl.ds(r0, rem), :],
                        outE_hbm.at[c, pl.ds(r0, rem), :])

    @pl.when(s == NS - 1)
    def _ctail():
        pltpu.sync_copy(accV.at[pl.ds(RPT * NS, NTAIL), :],
                        outV_hbm.at[c, pl.ds(RPT * NS, NTAIL), :])
        pltpu.sync_copy(accEW.at[pl.ds(RPT * NS, NTAIL), :],
                        outE_hbm.at[c, pl.ds(RPT * NS, NTAIL), :])


def _sc_call(e0, e1, ee, se, v, su, sv):
    mesh = plsc.VectorSubcoreMesh(core_axis_name="c", subcore_axis_name="s")
    kern = functools.partial(
        pl.kernel,
        mesh=mesh,
        out_type=(jax.ShapeDtypeStruct((NC, N, F), jnp.float32),
                  jax.ShapeDtypeStruct((NC, N, WEE), jnp.float32)),
        compiler_params=pltpu.CompilerParams(use_tc_tiling_on_sc=False,
                                             needs_layout_passes=False),
        scratch_types=[
            pltpu.VMEM((B,), jnp.int32),
            pltpu.VMEM((B,), jnp.int32),
            pltpu.VMEM((B, NR), jnp.float32),
            pltpu.VMEM((B,), jnp.float32),
            pltpu.VMEM((B,), jnp.float32),
            pltpu.VMEM((B,), jnp.float32),
            pltpu.VMEM((B,), jnp.float32),
            pltpu.VMEM((B, F), jnp.float32),
            pltpu.VMEM((B, WEE), jnp.float32),
            pltpu.VMEM_SHARED((N, F), jnp.float32),
            pltpu.VMEM_SHARED((N, WEE), jnp.float32),
            pltpu.SemaphoreType.DMA,
            pltpu.SemaphoreType.DMA,
            pltpu.SemaphoreType.DMA,
        ],
    )(_sc_body)
    return kern(e0, e1, ee, se, v, su, sv)


def _pre_body(x_ref, w_ref, o_ref):
    o_ref[...] = jnp.dot(x_ref[...], w_ref[...],
                         preferred_element_type=jnp.float32)


def _pre_call(x, w):
    blk = 1000
    return pl.pallas_call(
        _pre_body,
        grid=(N // blk,),
        in_specs=[
            pl.BlockSpec((blk, F), lambda i: (i, 0)),
            pl.BlockSpec((F, 3 * F), lambda i: (0, 0)),
        ],
        out_specs=pl.BlockSpec((blk, 3 * F), lambda i: (i, 0)),
        out_shape=jax.ShapeDtypeStruct((N, 3 * F), jnp.float32),
    )(x, w)


def _se_body(ee_ref, c2_ref, o_ref):
    o_ref[...] = jnp.sum(ee_ref[...] * c2_ref[...], axis=1)


def _se_call(ee, c2):
    blk = 512
    return pl.pallas_call(
        _se_body,
        grid=(E // blk,),
        in_specs=[
            pl.BlockSpec((blk, NR), lambda i: (i, 0)),
            pl.BlockSpec((1, NR), lambda i: (0, 0)),
        ],
        out_specs=pl.BlockSpec((blk,), lambda i: (i,)),
        out_shape=jax.ShapeDtypeStruct((E,), jnp.float32),
    )(ee, c2)


def _post_body(accv_ref, acce_ref, u_ref, a2t_ref, o_ref):
    accv = accv_ref[0] + accv_ref[1]
    accew = acce_ref[0] + acce_ref[1]
    acce = accew[:, :NR]
    rs = accew[:, NR:NR + 1]
    denom = jnp.where(rs == 0.0, 1e-12, rs)
    ind = jnp.where(rs > 0.0, 1.0, 0.0)
    h = u_ref[...] * ind + (accv + jnp.dot(acce, a2t_ref[...],
                                           preferred_element_type=jnp.float32)) / denom
    o_ref[...] = jnp.where(h > 0.0, h, jnp.exp(h) - 1.0)


def _post_call(accv, accew, u, a2t):
    blk = 1000
    return pl.pallas_call(
        _post_body,
        grid=(N // blk,),
        in_specs=[
            pl.BlockSpec((NC, blk, F), lambda i: (0, i, 0)),
            pl.BlockSpec((NC, blk, WEE), lambda i: (0, i, 0)),
            pl.BlockSpec((blk, F), lambda i: (i, 0)),
            pl.BlockSpec((NR, F), lambda i: (0, 0)),
        ],
        out_specs=pl.BlockSpec((blk, F), lambda i: (i, 0)),
        out_shape=jax.ShapeDtypeStruct((N, F), jnp.float32),
    )(accv, accew, u, a2t)


def kernel(input, edge, edge_embed, a, a_2):
    x = input
    a0t = a[:, :F].T
    a1t = a[:, F:2 * F].T
    a2t = a[:, 2 * F:].T
    c = (a_2 @ a)[0]
    cs = jnp.zeros((F, F), jnp.float32)
    cs = cs.at[:, 0].set(c[:F]).at[:, 1].set(c[F:2 * F])
    w_all = jnp.concatenate([a0t, a1t, cs], axis=1)

    xw = _pre_call(x, w_all)
    u = xw[:, :F]
    v = xw[:, F:2 * F]
    su = xw[:, 2 * F]
    sv = xw[:, 2 * F + 1]

    e0 = edge[0]
    e1 = edge[1]
    se = _se_call(edge_embed, c[2 * F:].reshape(1, NR))

    accv, accew = _sc_call(e0, e1, edge_embed, se, v, su, sv)
    return _post_call(accv, accew, u, a2t)


# speedup vs baseline: 3.7731x; 1.5604x over previous
import functools

import jax
import jax.numpy as jnp
from jax import lax
from jax.experimental import pallas as pl
from jax.experimental.pallas import tpu as pltpu
from jax.experimental.pallas import tpu_sc as plsc

N = 10000
E = 320000
F = 128
NR = 16
ALPHA = 0.2

NC = 2
NS = 16
NW = NC * NS
EPW = E // NW
B = 80
NB = EPW // B
WEE = 32
RPT = 624
NTAIL = N - RPT * NS


def _sc_body(e0_hbm, e1_hbm, ee_hbm, se_hbm, v_hbm, su_hbm, sv_hbm,
             outV_hbm, outE_hbm,
             e0_b, e1_b, ee_b, se_b, sug_b, svg_b, w_b, rows_b, wee_b,
             accV, accEW, sem, sem2, sem3):
    c = lax.axis_index("c")
    s = lax.axis_index("s")
    wid = c * NS + s

    zv = jnp.zeros((16,), jnp.float32)

    def _zrow(e, _):
        for j in range(F // 16):
            rows_b[e, pl.ds(j * 16, 16)] = zv
        for j in range(WEE // 16):
            wee_b[e, pl.ds(j * 16, 16)] = zv
        return 0
    lax.fori_loop(0, B, _zrow, 0)

    for k in range(RPT // B):
        r0 = s * RPT + k * B
        pltpu.sync_copy(rows_b, accV.at[pl.ds(r0, B), :])
        pltpu.sync_copy(wee_b, accEW.at[pl.ds(r0, B), :])
    r0 = s * RPT + (RPT // B) * B
    rem = RPT - (RPT // B) * B
    if rem:
        pltpu.sync_copy(rows_b.at[pl.ds(0, rem), :], accV.at[pl.ds(r0, rem), :])
        pltpu.sync_copy(wee_b.at[pl.ds(0, rem), :], accEW.at[pl.ds(r0, rem), :])

    @pl.when(s == NS - 1)
    def _ztail():
        pltpu.sync_copy(rows_b.at[pl.ds(0, NTAIL), :],
                        accV.at[pl.ds(RPT * NS, NTAIL), :])
        pltpu.sync_copy(wee_b.at[pl.ds(0, NTAIL), :],
                        accEW.at[pl.ds(RPT * NS, NTAIL), :])

    plsc.subcore_barrier()

    lanes = lax.iota(jnp.int32, 16)
    wcol = jnp.full((16,), NR, jnp.int32)

    def _batch(i, _):
        off = pl.multiple_of(wid * EPW + i * B, B)
        pltpu.sync_copy(e0_hbm.at[pl.ds(off, B)], e0_b)
        pltpu.sync_copy(e1_hbm.at[pl.ds(off, B)], e1_b)
        pltpu.sync_copy(ee_hbm.at[pl.ds(off, B), :], ee_b)
        pltpu.sync_copy(se_hbm.at[pl.ds(off, B)], se_b)
        gat = pltpu.async_copy(v_hbm.at[e1_b], rows_b, sem)
        gsu = pltpu.async_copy(su_hbm.at[e0_b], sug_b, sem2)
        gsv = pltpu.async_copy(sv_hbm.at[e1_b], svg_b, sem3)
        gsu.wait()
        gsv.wait()

        for j in range(B // 16):
            sl = pl.ds(j * 16, 16)
            z = sug_b[sl] + svg_b[sl] + se_b[sl]
            w = jnp.exp(jnp.minimum(-z, (-ALPHA) * z))
            w_b[sl] = w
            plsc.store_scatter(wee_b, [lanes + j * 16, wcol], w)

        gat.wait()

        def _edge(e, _):
            wv = plsc.load_gather(w_b, [jnp.full((16,), e, jnp.int32)])
            for j in range(F // 16):
                sl = pl.ds(j * 16, 16)
                rows_b[e, sl] = wv * rows_b[e, sl]
            wee_b[e, pl.ds(0, 16)] = wv * ee_b[e, :]
            return 0
        lax.fori_loop(0, B, _edge, 0)

        pltpu.sync_copy(rows_b, accV.at[e0_b], add=True)
        pltpu.sync_copy(wee_b, accEW.at[e0_b], add=True)
        return 0

    lax.fori_loop(0, NB, _batch, 0)

    plsc.subcore_barrier()

    for k in range(RPT // B):
        r0 = s * RPT + k * B
        pltpu.sync_copy(accV.at[pl.ds(r0, B), :], outV_hbm.at[c, pl.ds(r0, B), :])
        pltpu.sync_copy(accEW.at[pl.ds(r0, B), :], outE_hbm.at[c, pl.ds(r0, B), :])
    r0 = s * RPT + (RPT // B) * B
    if rem:
        pltpu.sync_copy(accV.at[pl.ds(r0, rem), :],
                        outV_hbm.at[c, pl.ds(r0, rem), :])
        pltpu.sync_copy(accEW.at[pl.ds(r0, rem), :],
                        outE_hbm.at[c, pl.ds(r0, rem), :])

    @pl.when(s == NS - 1)
    def _ctail():
        pltpu.sync_copy(accV.at[pl.ds(RPT * NS, NTAIL), :],
                        outV_hbm.at[c, pl.ds(RPT * NS, NTAIL), :])
        pltpu.sync_copy(accEW.at[pl.ds(RPT * NS, NTAIL), :],
                        outE_hbm.at[c, pl.ds(RPT * NS, NTAIL), :])


def _sc_call(e0, e1, ee, se, v, su, sv):
    mesh = plsc.VectorSubcoreMesh(core_axis_name="c", subcore_axis_name="s")
    kern = functools.partial(
        pl.kernel,
        mesh=mesh,
        out_type=(jax.ShapeDtypeStruct((NC, N, F), jnp.float32),
                  jax.ShapeDtypeStruct((NC, N, WEE), jnp.float32)),
        compiler_params=pltpu.CompilerParams(use_tc_tiling_on_sc=False,
                                             needs_layout_passes=False),
        scratch_types=[
            pltpu.VMEM((B,), jnp.int32),
            pltpu.VMEM((B,), jnp.int32),
            pltpu.VMEM((B, NR), jnp.float32),
            pltpu.VMEM((B,), jnp.float32),
            pltpu.VMEM((B,), jnp.float32),
            pltpu.VMEM((B,), jnp.float32),
            pltpu.VMEM((B,), jnp.float32),
            pltpu.VMEM((B, F), jnp.float32),
            pltpu.VMEM((B, WEE), jnp.float32),
            pltpu.VMEM_SHARED((N, F), jnp.float32),
            pltpu.VMEM_SHARED((N, WEE), jnp.float32),
            pltpu.SemaphoreType.DMA,
            pltpu.SemaphoreType.DMA,
            pltpu.SemaphoreType.DMA,
        ],
    )(_sc_body)
    return kern(e0, e1, ee, se, v, su, sv)


SEB = 250


def _pre_body(x_ref, wu_ref, wv_ref, cs_ref, eet_ref, c2_ref,
              u_ref, v_ref, suv_ref, se_ref):
    xb = x_ref[...]
    u_ref[...] = jnp.dot(xb, wu_ref[...], preferred_element_type=jnp.float32)
    v_ref[...] = jnp.dot(xb, wv_ref[...], preferred_element_type=jnp.float32)
    suv_ref[...] = jnp.dot(xb, cs_ref[...], preferred_element_type=jnp.float32)
    acc = eet_ref[0, 0] * c2_ref[0, 0]
    for k in range(1, NR):
        acc = acc + eet_ref[k, 0] * c2_ref[k, 0]
    se_ref[0] = acc


def _pre_call(x, wu, wv, cs, eet3, c2b):
    blk = 1000
    return pl.pallas_call(
        _pre_body,
        grid=(N // blk,),
        in_specs=[
            pl.BlockSpec((blk, F), lambda i: (i, 0)),
            pl.BlockSpec((F, F), lambda i: (0, 0)),
            pl.BlockSpec((F, F), lambda i: (0, 0)),
            pl.BlockSpec((F, F), lambda i: (0, 0)),
            pl.BlockSpec((NR, 1, SEB, F), lambda i: (0, i, 0, 0)),
            pl.BlockSpec((NR, 1, 1, F), lambda i: (0, 0, 0, 0)),
        ],
        out_specs=[
            pl.BlockSpec((blk, F), lambda i: (i, 0)),
            pl.BlockSpec((blk, F), lambda i: (i, 0)),
            pl.BlockSpec((blk, F), lambda i: (i, 0)),
            pl.BlockSpec((1, SEB, F), lambda i: (i, 0, 0)),
        ],
        out_shape=[
            jax.ShapeDtypeStruct((N, F), jnp.float32),
            jax.ShapeDtypeStruct((N, F), jnp.float32),
            jax.ShapeDtypeStruct((N, F), jnp.float32),
            jax.ShapeDtypeStruct((10, SEB, F), jnp.float32),
        ],
    )(x, wu, wv, cs, eet3, c2b)


def _post_body(accv_ref, acce_ref, u_ref, a2t_ref, o_ref):
    accv = accv_ref[0] + accv_ref[1]
    accew = acce_ref[0] + acce_ref[1]
    acce = accew[:, :NR]
    rs = accew[:, NR:NR + 1]
    denom = jnp.where(rs == 0.0, 1e-12, rs)
    ind = jnp.where(rs > 0.0, 1.0, 0.0)
    h = u_ref[...] * ind + (accv + jnp.dot(acce, a2t_ref[...],
                                           preferred_element_type=jnp.float32)) / denom
    o_ref[...] = jnp.where(h > 0.0, h, jnp.exp(h) - 1.0)


def _post_call(accv, accew, u, a2t):
    blk = 1000
    return pl.pallas_call(
        _post_body,
        grid=(N // blk,),
        in_specs=[
            pl.BlockSpec((NC, blk, F), lambda i: (0, i, 0)),
            pl.BlockSpec((NC, blk, WEE), lambda i: (0, i, 0)),
            pl.BlockSpec((blk, F), lambda i: (i, 0)),
            pl.BlockSpec((NR, F), lambda i: (0, 0)),
        ],
        out_specs=pl.BlockSpec((blk, F), lambda i: (i, 0)),
        out_shape=jax.ShapeDtypeStruct((N, F), jnp.float32),
    )(accv, accew, u, a2t)


def kernel(input, edge, edge_embed, a, a_2):
    x = input
    a0t = a[:, :F].T
    a1t = a[:, F:2 * F].T
    a2t = a[:, 2 * F:].T
    c = (a_2 @ a)[0]
    cs = jnp.zeros((F, F), jnp.float32)
    cs = cs.at[:, 0].set(c[:F]).at[:, 1].set(c[F:2 * F])
    eet3 = edge_embed.T.reshape(NR, 10, SEB, F)
    c2b = jnp.broadcast_to(c[2 * F:].reshape(NR, 1, 1, 1), (NR, 1, 1, F))

    u, v, suv, se2 = _pre_call(x, a0t, a1t, cs, eet3, c2b)
    su = suv[:, 0]
    sv = suv[:, 1]
    se = se2.reshape(E)

    e0 = edge[0]
    e1 = edge[1]

    accv, accew = _sc_call(e0, e1, edge_embed, se, v, su, sv)
    return _post_call(accv, accew, u, a2t)


# speedup vs baseline: 4.8949x; 1.2973x over previous
import functools

import jax
import jax.numpy as jnp
from jax import lax
from jax.experimental import pallas as pl
from jax.experimental.pallas import tpu as pltpu
from jax.experimental.pallas import tpu_sc as plsc

N = 10000
E = 320000
F = 128
NR = 16
ALPHA = 0.2

NC = 2
NS = 16
NW = NC * NS
EPW = E // NW
B = 80
NB = EPW // B
WEE = 32
RPT = 624
NTAIL = N - RPT * NS


def _sc_body(e0_hbm, e1_hbm, ee_hbm, se_hbm, v_hbm, su_hbm, sv_hbm,
             outV_hbm, outE_hbm,
             e0_b4, e1_b4, se_b4, ee_b2, sug_b2, svg_b2, w_b, rows_b2, wee_b2,
             accV, accEW, sem_sl, sem_ee, sem_g, sem_sc):
    c = lax.axis_index("c")
    s = lax.axis_index("s")
    wid = c * NS + s
    base = wid * EPW

    zv = jnp.zeros((16,), jnp.float32)

    def _zrow(e, _):
        for j in range(F // 16):
            rows_b2[0, e, pl.ds(j * 16, 16)] = zv
        for j in range(WEE // 16):
            wee_b2[0, e, pl.ds(j * 16, 16)] = zv
            wee_b2[1, e, pl.ds(j * 16, 16)] = zv
        return 0
    lax.fori_loop(0, B, _zrow, 0)

    for k in range(RPT // B):
        r0 = s * RPT + k * B
        pltpu.sync_copy(rows_b2.at[0], accV.at[pl.ds(r0, B), :])
        pltpu.sync_copy(wee_b2.at[0], accEW.at[pl.ds(r0, B), :])
    r0 = s * RPT + (RPT // B) * B
    rem = RPT - (RPT // B) * B
    if rem:
        pltpu.sync_copy(rows_b2.at[0, pl.ds(0, rem), :],
                        accV.at[pl.ds(r0, rem), :])
        pltpu.sync_copy(wee_b2.at[0, pl.ds(0, rem), :],
                        accEW.at[pl.ds(r0, rem), :])

    @pl.when(s == NS - 1)
    def _ztail():
        pltpu.sync_copy(rows_b2.at[0, pl.ds(0, NTAIL), :],
                        accV.at[pl.ds(RPT * NS, NTAIL), :])
        pltpu.sync_copy(wee_b2.at[0, pl.ds(0, NTAIL), :],
                        accEW.at[pl.ds(RPT * NS, NTAIL), :])

    plsc.subcore_barrier()

    lanes = lax.iota(jnp.int32, 16)
    wcol = jnp.full((16,), NR, jnp.int32)

    def _sl_descs(bi):
        sp = bi & 3
        off = pl.multiple_of(base + bi * B, B)
        return (
            pltpu.make_async_copy(e0_hbm.at[pl.ds(off, B)], e0_b4.at[sp], sem_sl),
            pltpu.make_async_copy(e1_hbm.at[pl.ds(off, B)], e1_b4.at[sp], sem_sl),
            pltpu.make_async_copy(se_hbm.at[pl.ds(off, B)], se_b4.at[sp], sem_sl),
        )

    def _g_descs(bi):
        sp = bi & 3
        p = bi & 1
        return (
            pltpu.make_async_copy(v_hbm.at[e1_b4.at[sp]], rows_b2.at[p], sem_g),
            pltpu.make_async_copy(su_hbm.at[e0_b4.at[sp]], sug_b2.at[p], sem_g),
            pltpu.make_async_copy(sv_hbm.at[e1_b4.at[sp]], svg_b2.at[p], sem_g),
        )

    def _ee_desc(bi):
        p = bi & 1
        off = pl.multiple_of(base + bi * B, B)
        return pltpu.make_async_copy(ee_hbm.at[pl.ds(off, B), :],
                                     ee_b2.at[p], sem_ee)

    def _issue_gathers(bi):
        _ee_desc(bi).start()
        for d in _g_descs(bi):
            d.start()

    def _sc_descs(bi):
        sp = bi & 3
        p = bi & 1
        return (
            pltpu.make_async_copy(rows_b2.at[p], accV.at[e0_b4.at[sp]], sem_sc),
            pltpu.make_async_copy(wee_b2.at[p], accEW.at[e0_b4.at[sp]], sem_sc),
        )

    for d in _sl_descs(0):
        d.start()
    for d in _sl_descs(1):
        d.start()
    for d in _sl_descs(0):
        d.wait()
    _issue_gathers(0)

    def _batch(i, _):
        p = i & 1

        @pl.when(i + 1 < NB)
        def _():
            for d in _sl_descs(i + 1):
                d.wait()

        @pl.when(i >= 1)
        def _():
            for d in _sc_descs(i - 1):
                d.wait()

        @pl.when(i + 1 < NB)
        def _():
            _issue_gathers(i + 1)

        _ee_desc(i).wait()
        for d in _g_descs(i):
            d.wait()

        for j in range(B // 16):
            sl = pl.ds(j * 16, 16)
            z = sug_b2[p, sl] + svg_b2[p, sl] + se_b4[i & 3, sl]
            w = jnp.exp(jnp.minimum(-z, (-ALPHA) * z))
            w_b[sl] = w
            plsc.store_scatter(wee_b2, [jnp.full((16,), p, jnp.int32),
                                        lanes + j * 16, wcol], w)

        def _edge(e, _):
            wv = plsc.load_gather(w_b, [jnp.full((16,), e, jnp.int32)])
            for j in range(F // 16):
                sl = pl.ds(j * 16, 16)
                rows_b2[p, e, sl] = wv * rows_b2[p, e, sl]
            wee_b2[p, e, pl.ds(0, 16)] = wv * ee_b2[p, e, :]
            return 0
        lax.fori_loop(0, B, _edge, 0)

        for d in _sc_descs(i):
            d.start(add=True)

        @pl.when(i + 2 < NB)
        def _():
            for d in _sl_descs(i + 2):
                d.start()
        return 0

    lax.fori_loop(0, NB, _batch, 0)
    for d in _sc_descs(NB - 1):
        d.wait()

    plsc.subcore_barrier()

    for k in range(RPT // B):
        r0 = s * RPT + k * B
        pltpu.sync_copy(accV.at[pl.ds(r0, B), :], outV_hbm.at[c, pl.ds(r0, B), :])
        pltpu.sync_copy(accEW.at[pl.ds(r0, B), :], outE_hbm.at[c, pl.ds(r0, B), :])
    r0 = s * RPT + (RPT // B) * B
    if rem:
        pltpu.sync_copy(accV.at[pl.ds(r0, rem), :],
                        outV_hbm.at[c, pl.ds(r0, rem), :])
        pltpu.sync_copy(accEW.at[pl.ds(r0, rem), :],
                        outE_hbm.at[c, pl.ds(r0, rem), :])

    @pl.when(s == NS - 1)
    def _ctail():
        pltpu.sync_copy(accV.at[pl.ds(RPT * NS, NTAIL), :],
                        outV_hbm.at[c, pl.ds(RPT * NS, NTAIL), :])
        pltpu.sync_copy(accEW.at[pl.ds(RPT * NS, NTAIL), :],
                        outE_hbm.at[c, pl.ds(RPT * NS, NTAIL), :])


def _sc_call(e0, e1, ee, se, v, su, sv):
    mesh = plsc.VectorSubcoreMesh(core_axis_name="c", subcore_axis_name="s")
    kern = functools.partial(
        pl.kernel,
        mesh=mesh,
        out_type=(jax.ShapeDtypeStruct((NC, N, F), jnp.float32),
                  jax.ShapeDtypeStruct((NC, N, WEE), jnp.float32)),
        compiler_params=pltpu.CompilerParams(use_tc_tiling_on_sc=False,
                                             needs_layout_passes=False),
        scratch_types=[
            pltpu.VMEM((4, B), jnp.int32),
            pltpu.VMEM((4, B), jnp.int32),
            pltpu.VMEM((4, B), jnp.float32),
            pltpu.VMEM((2, B, NR), jnp.float32),
            pltpu.VMEM((2, B), jnp.float32),
            pltpu.VMEM((2, B), jnp.float32),
            pltpu.VMEM((B,), jnp.float32),
            pltpu.VMEM((2, B, F), jnp.float32),
            pltpu.VMEM((2, B, WEE), jnp.float32),
            pltpu.VMEM_SHARED((N, F), jnp.float32),
            pltpu.VMEM_SHARED((N, WEE), jnp.float32),
            pltpu.SemaphoreType.DMA,
            pltpu.SemaphoreType.DMA,
            pltpu.SemaphoreType.DMA,
            pltpu.SemaphoreType.DMA,
        ],
    )(_sc_body)
    return kern(e0, e1, ee, se, v, su, sv)


SEB = E // 10


def _pre_body(x_ref, wu_ref, wv_ref, cs_ref, ee_ref, c2_ref,
              u_ref, v_ref, suv_ref, se_ref):
    i = pl.program_id(0)
    xb = x_ref[...]
    u_ref[...] = jnp.dot(xb, wu_ref[...], preferred_element_type=jnp.float32)
    v_ref[...] = jnp.dot(xb, wv_ref[...], preferred_element_type=jnp.float32)
    suv_ref[...] = jnp.dot(xb, cs_ref[...], preferred_element_type=jnp.float32)
    seb = jnp.sum(ee_ref[...] * c2_ref[...], axis=1)
    se_ref[pl.ds(pl.multiple_of(i * SEB, 128), SEB)] = seb


def _pre_call(x, wu, wv, cs, ee, c2b):
    blk = 1000
    return pl.pallas_call(
        _pre_body,
        grid=(N // blk,),
        in_specs=[
            pl.BlockSpec((blk, F), lambda i: (i, 0)),
            pl.BlockSpec((F, F), lambda i: (0, 0)),
            pl.BlockSpec((F, F), lambda i: (0, 0)),
            pl.BlockSpec((F, F), lambda i: (0, 0)),
            pl.BlockSpec((SEB, NR), lambda i: (i, 0)),
            pl.BlockSpec((1, NR), lambda i: (0, 0)),
        ],
        out_specs=[
            pl.BlockSpec((blk, F), lambda i: (i, 0)),
            pl.BlockSpec((blk, F), lambda i: (i, 0)),
            pl.BlockSpec((blk, F), lambda i: (i, 0)),
            pl.BlockSpec((E,), lambda i: (0,)),
        ],
        out_shape=[
            jax.ShapeDtypeStruct((N, F), jnp.float32),
            jax.ShapeDtypeStruct((N, F), jnp.float32),
            jax.ShapeDtypeStruct((N, F), jnp.float32),
            jax.ShapeDtypeStruct((E,), jnp.float32),
        ],
    )(x, wu, wv, cs, ee, c2b)


def _post_body(accv_ref, acce_ref, u_ref, a2t_ref, o_ref):
    accv = accv_ref[0] + accv_ref[1]
    accew = acce_ref[0] + acce_ref[1]
    acce = accew[:, :NR]
    rs = accew[:, NR:NR + 1]
    denom = jnp.where(rs == 0.0, 1e-12, rs)
    ind = jnp.where(rs > 0.0, 1.0, 0.0)
    h = u_ref[...] * ind + (accv + jnp.dot(acce, a2t_ref[...],
                                           preferred_element_type=jnp.float32)) / denom
    o_ref[...] = jnp.where(h > 0.0, h, jnp.exp(h) - 1.0)


def _post_call(accv, accew, u, a2t):
    blk = 1000
    return pl.pallas_call(
        _post_body,
        grid=(N // blk,),
        in_specs=[
            pl.BlockSpec((NC, blk, F), lambda i: (0, i, 0)),
            pl.BlockSpec((NC, blk, WEE), lambda i: (0, i, 0)),
            pl.BlockSpec((blk, F), lambda i: (i, 0)),
            pl.BlockSpec((NR, F), lambda i: (0, 0)),
        ],
        out_specs=pl.BlockSpec((blk, F), lambda i: (i, 0)),
        out_shape=jax.ShapeDtypeStruct((N, F), jnp.float32),
    )(accv, accew, u, a2t)


def kernel(input, edge, edge_embed, a, a_2):
    x = input
    a0t = a[:, :F].T
    a1t = a[:, F:2 * F].T
    a2t = a[:, 2 * F:].T
    c = (a_2 @ a)[0]
    cs = jnp.zeros((F, F), jnp.float32)
    cs = cs.at[:, 0].set(c[:F]).at[:, 1].set(c[F:2 * F])
    c2b = c[2 * F:].reshape(1, NR)

    u, v, suv, se = _pre_call(x, a0t, a1t, cs, edge_embed, c2b)
    su = suv[:, 0]
    sv = suv[:, 1]

    e0 = edge[0]
    e1 = edge[1]

    accv, accew = _sc_call(e0, e1, edge_embed, se, v, su, sv)
    return _post_call(accv, accew, u, a2t)


# speedup vs baseline: 5.7457x; 1.1738x over previous
import functools

import jax
import jax.numpy as jnp
from jax import lax
from jax.experimental import pallas as pl
from jax.experimental.pallas import tpu as pltpu
from jax.experimental.pallas import tpu_sc as plsc

N = 10000
E = 320000
F = 128
NR = 16
ALPHA = 0.2

NC = 2
NS = 16
NW = NC * NS
EPW = E // NW
B = 80
NB = EPW // B
WEE = 32
RPT = 624
NTAIL = N - RPT * NS


def _sc_body(e0_hbm, e1_hbm, ee_hbm, sp_hbm, v_hbm, su_hbm, sv_hbm,
             outV_hbm, outE_hbm,
             e0_b4, e1_b4, sp_b4, ee_b2, sug_b2, svg_b2, w_b, rows_b2, wee_b2,
             accV, accEW, sem_sl, sem_ee, sem_g, sem_sc):
    c = lax.axis_index("c")
    s = lax.axis_index("s")
    wid = c * NS + s
    base = wid * EPW

    zv = jnp.zeros((16,), jnp.float32)

    def _zrow(e, _):
        for j in range(F // 16):
            rows_b2[0, e, pl.ds(j * 16, 16)] = zv
        for j in range(WEE // 16):
            wee_b2[0, e, pl.ds(j * 16, 16)] = zv
            wee_b2[1, e, pl.ds(j * 16, 16)] = zv
        return 0
    lax.fori_loop(0, B, _zrow, 0)

    for k in range(RPT // B):
        r0 = s * RPT + k * B
        pltpu.sync_copy(rows_b2.at[0], accV.at[pl.ds(r0, B), :])
        pltpu.sync_copy(wee_b2.at[0], accEW.at[pl.ds(r0, B), :])
    r0 = s * RPT + (RPT // B) * B
    rem = RPT - (RPT // B) * B
    if rem:
        pltpu.sync_copy(rows_b2.at[0, pl.ds(0, rem), :],
                        accV.at[pl.ds(r0, rem), :])
        pltpu.sync_copy(wee_b2.at[0, pl.ds(0, rem), :],
                        accEW.at[pl.ds(r0, rem), :])

    @pl.when(s == NS - 1)
    def _ztail():
        pltpu.sync_copy(rows_b2.at[0, pl.ds(0, NTAIL), :],
                        accV.at[pl.ds(RPT * NS, NTAIL), :])
        pltpu.sync_copy(wee_b2.at[0, pl.ds(0, NTAIL), :],
                        accEW.at[pl.ds(RPT * NS, NTAIL), :])

    plsc.subcore_barrier()


    def _sl_descs(bi):
        sp = bi & 3
        off = pl.multiple_of(base + bi * B, B)
        return (
            pltpu.make_async_copy(e0_hbm.at[pl.ds(off, B)], e0_b4.at[sp], sem_sl),
            pltpu.make_async_copy(e1_hbm.at[pl.ds(off, B)], e1_b4.at[sp], sem_sl),
        )

    def _g_descs(bi):
        sp = bi & 3
        p = bi & 1
        return (
            pltpu.make_async_copy(v_hbm.at[e1_b4.at[sp]], rows_b2.at[p], sem_g),
            pltpu.make_async_copy(su_hbm.at[e0_b4.at[sp]], sug_b2.at[p], sem_g),
            pltpu.make_async_copy(sv_hbm.at[e1_b4.at[sp]], svg_b2.at[p], sem_g),
        )

    def _ee_descs(bi):
        p = bi & 1
        off = pl.multiple_of(base + bi * B, B)
        off8 = pl.multiple_of((base + bi * B) // 8, 2)
        return (
            pltpu.make_async_copy(ee_hbm.at[pl.ds(off, B), :],
                                  ee_b2.at[p], sem_ee),
            pltpu.make_async_copy(sp_hbm.at[pl.ds(off8, B // 8), pl.ds(0, 64)],
                                  sp_b4.at[pl.ds(p * (B // 8), B // 8), :],
                                  sem_ee),
        )

    def _issue_gathers(bi):
        for d in _ee_descs(bi):
            d.start()
        for d in _g_descs(bi):
            d.start()

    def _sc_descs(bi):
        sp = bi & 3
        p = bi & 1
        return (
            pltpu.make_async_copy(rows_b2.at[p], accV.at[e0_b4.at[sp]], sem_sc),
            pltpu.make_async_copy(wee_b2.at[p], accEW.at[e0_b4.at[sp]], sem_sc),
        )

    for d in _sl_descs(0):
        d.start()
    for d in _sl_descs(1):
        d.start()
    for d in _sl_descs(0):
        d.wait()
    _issue_gathers(0)

    def _batch(i, _):
        p = i & 1

        @pl.when(i + 1 < NB)
        def _():
            for d in _sl_descs(i + 1):
                d.wait()

        @pl.when(i >= 1)
        def _():
            for d in _sc_descs(i - 1):
                d.wait()

        @pl.when(i + 1 < NB)
        def _():
            _issue_gathers(i + 1)

        for d in _ee_descs(i):
            d.wait()
        for d in _g_descs(i):
            d.wait()

        lanes = lax.iota(jnp.int32, 16)
        wcol = jnp.full((16,), NR, jnp.int32)
        pfull = jnp.full((16,), p, jnp.int32)
        sprow0 = p * (B // 8)
        for j in range(B // 16):
            sl = pl.ds(j * 16, 16)
            rows16 = lanes + j * 16
            sev = plsc.load_gather(
                sp_b4,
                [(lanes >> 3) + (sprow0 + 2 * j), lanes & 7])
            z = sug_b2[p, sl] + svg_b2[p, sl] + sev
            w = jnp.exp(jnp.minimum(-z, (-ALPHA) * z))
            w_b[sl] = w
            plsc.store_scatter(wee_b2, [pfull, rows16, wcol], w)

        def _edge(e, _):
            wv = plsc.load_gather(w_b, [jnp.full((16,), e, jnp.int32)])
            for j in range(F // 16):
                sl = pl.ds(j * 16, 16)
                rows_b2[p, e, sl] = wv * rows_b2[p, e, sl]
            wee_b2[p, e, pl.ds(0, 16)] = wv * ee_b2[p, e, :]
            return 0
        lax.fori_loop(0, B, _edge, 0)

        for d in _sc_descs(i):
            d.start(add=True)

        @pl.when(i + 2 < NB)
        def _():
            for d in _sl_descs(i + 2):
                d.start()
        return 0

    lax.fori_loop(0, NB, _batch, 0)
    for d in _sc_descs(NB - 1):
        d.wait()

    plsc.subcore_barrier()

    for k in range(RPT // B):
        r0 = s * RPT + k * B
        pltpu.sync_copy(accV.at[pl.ds(r0, B), :], outV_hbm.at[c, pl.ds(r0, B), :])
        pltpu.sync_copy(accEW.at[pl.ds(r0, B), :], outE_hbm.at[c, pl.ds(r0, B), :])
    r0 = s * RPT + (RPT // B) * B
    if rem:
        pltpu.sync_copy(accV.at[pl.ds(r0, rem), :],
                        outV_hbm.at[c, pl.ds(r0, rem), :])
        pltpu.sync_copy(accEW.at[pl.ds(r0, rem), :],
                        outE_hbm.at[c, pl.ds(r0, rem), :])

    @pl.when(s == NS - 1)
    def _ctail():
        pltpu.sync_copy(accV.at[pl.ds(RPT * NS, NTAIL), :],
                        outV_hbm.at[c, pl.ds(RPT * NS, NTAIL), :])
        pltpu.sync_copy(accEW.at[pl.ds(RPT * NS, NTAIL), :],
                        outE_hbm.at[c, pl.ds(RPT * NS, NTAIL), :])


def _sc_call(e0, e1, ee, se_pack, v, su, sv):
    mesh = plsc.VectorSubcoreMesh(core_axis_name="c", subcore_axis_name="s")
    kern = functools.partial(
        pl.kernel,
        mesh=mesh,
        out_type=(jax.ShapeDtypeStruct((NC, N, F), jnp.float32),
                  jax.ShapeDtypeStruct((NC, N, WEE), jnp.float32)),
        compiler_params=pltpu.CompilerParams(use_tc_tiling_on_sc=False,
                                             needs_layout_passes=False),
        scratch_types=[
            pltpu.VMEM((4, B), jnp.int32),
            pltpu.VMEM((4, B), jnp.int32),
            pltpu.VMEM((2 * (B // 8), 64), jnp.float32),
            pltpu.VMEM((2, B, NR), jnp.float32),
            pltpu.VMEM((2, B), jnp.float32),
            pltpu.VMEM((2, B), jnp.float32),
            pltpu.VMEM((B,), jnp.float32),
            pltpu.VMEM((2, B, F), jnp.float32),
            pltpu.VMEM((2, B, WEE), jnp.float32),
            pltpu.VMEM_SHARED((N, F), jnp.float32),
            pltpu.VMEM_SHARED((N, WEE), jnp.float32),
            pltpu.SemaphoreType.DMA,
            pltpu.SemaphoreType.DMA,
            pltpu.SemaphoreType.DMA,
            pltpu.SemaphoreType.DMA,
        ],
    )(_sc_body)
    return kern(e0, e1, ee, se_pack, v, su, sv)


E8 = E // 8


def _pre_body(x_ref, wu_ref, wv_ref, cs_ref, eel_ref, d_ref,
              u_ref, v_ref, suv_ref, sp_ref):
    xb = x_ref[...]
    u_ref[...] = jnp.dot(xb, wu_ref[...], preferred_element_type=jnp.float32)
    v_ref[...] = jnp.dot(xb, wv_ref[...], preferred_element_type=jnp.float32)
    suv_ref[...] = jnp.dot(xb, cs_ref[...], preferred_element_type=jnp.float32)
    sp_ref[...] = jnp.dot(eel_ref[...], d_ref[...],
                          preferred_element_type=jnp.float32)


def _pre_call(x, wu, wv, cs, eel, d):
    blk = 1000
    return pl.pallas_call(
        _pre_body,
        grid=(N // blk,),
        in_specs=[
            pl.BlockSpec((blk, F), lambda i: (i, 0)),
            pl.BlockSpec((F, F), lambda i: (0, 0)),
            pl.BlockSpec((F, F), lambda i: (0, 0)),
            pl.BlockSpec((F, F), lambda i: (0, 0)),
            pl.BlockSpec((E8 // 10, F), lambda i: (i, 0)),
            pl.BlockSpec((F, F), lambda i: (0, 0)),
        ],
        out_specs=[
            pl.BlockSpec((blk, F), lambda i: (i, 0)),
            pl.BlockSpec((blk, F), lambda i: (i, 0)),
            pl.BlockSpec((blk, F), lambda i: (i, 0)),
            pl.BlockSpec((E8 // 10, F), lambda i: (i, 0)),
        ],
        out_shape=[
            jax.ShapeDtypeStruct((N, F), jnp.float32),
            jax.ShapeDtypeStruct((N, F), jnp.float32),
            jax.ShapeDtypeStruct((N, F), jnp.float32),
            jax.ShapeDtypeStruct((E8, F), jnp.float32),
        ],
    )(x, wu, wv, cs, eel, d)


def _post_body(accv_ref, acce_ref, u_ref, a2t_ref, o_ref):
    accv = accv_ref[0] + accv_ref[1]
    accew = acce_ref[0] + acce_ref[1]
    acce = accew[:, :NR]
    rs = accew[:, NR:NR + 1]
    denom = jnp.where(rs == 0.0, 1e-12, rs)
    ind = jnp.where(rs > 0.0, 1.0, 0.0)
    h = u_ref[...] * ind + (accv + jnp.dot(acce, a2t_ref[...],
                                           preferred_element_type=jnp.float32)) / denom
    o_ref[...] = jnp.where(h > 0.0, h, jnp.exp(h) - 1.0)


def _post_call(accv, accew, u, a2t):
    blk = 1000
    return pl.pallas_call(
        _post_body,
        grid=(N // blk,),
        in_specs=[
            pl.BlockSpec((NC, blk, F), lambda i: (0, i, 0)),
            pl.BlockSpec((NC, blk, WEE), lambda i: (0, i, 0)),
            pl.BlockSpec((blk, F), lambda i: (i, 0)),
            pl.BlockSpec((NR, F), lambda i: (0, 0)),
        ],
        out_specs=pl.BlockSpec((blk, F), lambda i: (i, 0)),
        out_shape=jax.ShapeDtypeStruct((N, F), jnp.float32),
    )(accv, accew, u, a2t)


def kernel(input, edge, edge_embed, a, a_2):
    x = input
    a0t = a[:, :F].T
    a1t = a[:, F:2 * F].T
    a2t = a[:, 2 * F:].T
    c = (a_2 @ a)[0]
    cs = jnp.zeros((F, F), jnp.float32)
    cs = cs.at[:, 0].set(c[:F]).at[:, 1].set(c[F:2 * F])
    c2 = c[2 * F:]
    d = jnp.zeros((F, F), jnp.float32)
    d = d.at[jnp.arange(F), jnp.arange(F) // NR].set(jnp.tile(c2, 8))
    eel = edge_embed.reshape(E8, F)

    u, v, suv, se_pack = _pre_call(x, a0t, a1t, cs, eel, d)
    su = suv[:, 0]
    sv = suv[:, 1]

    e0 = edge[0]
    e1 = edge[1]

    accv, accew = _sc_call(e0, e1, edge_embed, se_pack, v, su, sv)
    return _post_call(accv, accew, u, a2t)
